# baseline (speedup 1.0000x reference)
import functools

import jax
import jax.numpy as jnp
from jax import lax
from jax.experimental import pallas as pl
from jax.experimental.pallas import tpu as pltpu
from jax.experimental.pallas import tpu_sc as plsc

_NBUF = 2
_LDBATCH = 8


@functools.lru_cache(maxsize=None)
def _make_sc_kernel(n, d, v, chunk, nbuf):
    info = plsc.get_sparse_core_info()
    nc, ns = info.num_cores, info.num_subcores
    nw = nc * ns
    per_w = n // nw
    assert per_w * nw == n
    n_chunks = per_w // chunk
    assert n_chunks * chunk == per_w and n_chunks % nbuf == 0
    n_groups = n_chunks // nbuf
    lanes = info.num_lanes
    assert d % (lanes * _LDBATCH) == 0
    mesh = plsc.VectorSubcoreMesh(core_axis_name="c", subcore_axis_name="s")

    @functools.partial(
        pl.kernel,
        mesh=mesh,
        out_type=jax.ShapeDtypeStruct((n, d), jnp.float32),
        scratch_types=(
            [pltpu.VMEM((per_w,), jnp.int32),
             pltpu.VMEM((v, d), jnp.float32)]
            + [pltpu.VMEM((chunk, d), jnp.float32) for _ in range(nbuf)]
            + [pltpu.SemaphoreType.DMA for _ in range(nbuf)]
        ),
    )
    def k(idx_hbm, table_hbm, out_hbm, idx_all, table_v, *bufs_and_sems):
        rows = bufs_and_sems[:nbuf]
        ssem = bufs_and_sems[nbuf:2 * nbuf]
        wid = lax.axis_index("s") * nc + lax.axis_index("c")
        base = wid * per_w

        pltpu.sync_copy(table_hbm, table_v)
        pltpu.sync_copy(idx_hbm.at[pl.ds(base, per_w)], idx_all)

        def expand(c, b):
            def group_body(i0, carry):
                riv = idx_all[pl.ds(c * chunk + i0, lanes)]
                for l in range(lanes):
                    r = riv[l]
                    for jb in range(0, d // lanes, _LDBATCH):
                        vals = [table_v[r, pl.ds((jb + j) * lanes, lanes)]
                                for j in range(_LDBATCH)]
                        for j in range(_LDBATCH):
                            rows[b][i0 + l,
                                    pl.ds((jb + j) * lanes, lanes)] = vals[j]
                return carry
            lax.fori_loop(0, chunk // lanes,
                          lambda i, cc: group_body(i * lanes, cc), 0)

        def scat(c, b):
            pltpu.async_copy(
                rows[b], out_hbm.at[pl.ds(base + c * chunk, chunk)], ssem[b])

        def wait_scat(c, b):
            pltpu.make_async_copy(
                rows[b], out_hbm.at[pl.ds(base + c * chunk, chunk)],
                ssem[b]).wait()

        for b in range(nbuf):
            expand(b, b)
            scat(b, b)

        def body(g, carry):
            plsc.subcore_barrier()
            c0 = (g + 1) * nbuf
            for b in range(nbuf):
                c = c0 + b
                wait_scat(c - nbuf, b)
                expand(c, b)
                scat(c, b)
            return carry

        lax.fori_loop(0, n_groups - 1, body, 0)
        for b in range(nbuf):
            wait_scat(n_chunks - nbuf + b, b)

    return k


def kernel(x, weight):
    orig_shape = x.shape
    v, d = weight.shape
    flat = x.reshape(-1).astype(jnp.int32)
    n = flat.shape[0]
    out = _make_sc_kernel(n, d, v, 64, _NBUF)(flat, weight)
    return out.reshape(*orig_shape, d)


# speedup vs baseline: 1.3090x; 1.3090x over previous
import jax
import jax.numpy as jnp
from jax.experimental import pallas as pl


_EMB_PAD = 16


def _tc_body(idx_ref, w_ref, o_ref):
    idx = idx_ref[0, 0, :]
    onehot = (idx[:, None] == jax.lax.broadcasted_iota(
        jnp.int32, (idx.shape[0], _EMB_PAD), 1)).astype(jnp.float32)
    o_ref[...] = jnp.dot(onehot, w_ref[...], preferred_element_type=jnp.float32)


def kernel(x, weight):
    orig_shape = x.shape
    num_emb, d = weight.shape
    flat = x.reshape(-1).astype(jnp.int32)
    n = flat.shape[0]
    block = 1024
    nb = n // block
    assert nb * block == n
    idx3 = flat.reshape(nb, 1, block)
    wpad = jnp.zeros((_EMB_PAD, d), weight.dtype).at[:num_emb].set(weight)
    out = pl.pallas_call(
        _tc_body,
        grid=(nb,),
        in_specs=[
            pl.BlockSpec((1, 1, block), lambda i: (i, 0, 0)),
            pl.BlockSpec((_EMB_PAD, d), lambda i: (0, 0)),
        ],
        out_specs=pl.BlockSpec((block, d), lambda i: (i, 0)),
        out_shape=jax.ShapeDtypeStruct((n, d), jnp.float32),
    )(idx3, wpad)
    return out.reshape(*orig_shape, d)
